# baseline (speedup 1.0000x reference)
import functools

import jax
import jax.numpy as jnp
from jax import lax
from jax.experimental import pallas as pl
from jax.experimental.pallas import tpu as pltpu
from jax.experimental.pallas import tpu_sc as plsc

BATCH = 4096
HIST = 50
DIM = 128
NC = 2
NS = 16
NW = NC * NS
N_IDX = BATCH * HIST
PER_W = N_IDX // NW
CHUNK = 80
NCH = PER_W // CHUNK


VOCAB = 1000
NBUF = 8
STAGE_T = 16
STAGE_R = 64


def _make_gather():
    mesh = plsc.VectorSubcoreMesh(core_axis_name="c", subcore_axis_name="s")

    @functools.partial(
        pl.kernel,
        mesh=mesh,
        out_type=jax.ShapeDtypeStruct((N_IDX, DIM), jnp.float32),
        scratch_types=[
            pltpu.VMEM((NCH, CHUNK), jnp.int32),
        ] + [pltpu.VMEM((CHUNK, DIM), jnp.float32) for _ in range(NBUF)] + [
            pltpu.VMEM_SHARED((VOCAB, DIM), jnp.float32),
        ] + [pltpu.SemaphoreType.DMA for _ in range(2 * NBUF)],
    )
    def gather_k(idx_hbm, table_hbm, out_hbm, idx_v, *rest):
        bufs = rest[:NBUF]
        tab_s = rest[NBUF]
        gs = rest[NBUF + 1:2 * NBUF + 1]
        os_ = rest[2 * NBUF + 1:]
        sid = lax.axis_index("s")
        wid = sid * NC + lax.axis_index("c")
        base = wid * PER_W

        for t in range(STAGE_T):
            nrows = min(STAGE_R, VOCAB - t * STAGE_R)

            @pl.when(sid == t)
            def _(t=t, nrows=nrows):
                pltpu.sync_copy(table_hbm.at[pl.ds(t * STAGE_R, nrows)],
                                tab_s.at[pl.ds(t * STAGE_R, nrows)])

        pltpu.sync_copy(idx_hbm.at[wid], idx_v)
        plsc.subcore_barrier()

        def out_slice(c):
            return out_hbm.at[pl.ds(base + c * CHUNK, CHUNK)]

        for bx in range(NBUF):
            pltpu.async_copy(tab_s.at[idx_v.at[bx]], bufs[bx], gs[bx])

        def step(i, carry):
            c0 = NBUF * i
            for bx in range(NBUF):
                pltpu.make_async_copy(
                    tab_s.at[idx_v.at[c0]], bufs[bx], gs[bx]).wait()
                pltpu.async_copy(bufs[bx], out_slice(c0 + bx), os_[bx])

            @pl.when(i < NCH // NBUF - 1)
            def _():
                for bx in range(NBUF):
                    pltpu.make_async_copy(
                        bufs[bx], out_slice(c0 + bx), os_[bx]).wait()
                    pltpu.async_copy(
                        tab_s.at[idx_v.at[c0 + NBUF + bx]], bufs[bx], gs[bx])

            return carry

        lax.fori_loop(0, NCH // NBUF, step, 0, unroll=False)

        for bx in range(NBUF):
            pltpu.make_async_copy(
                bufs[bx], out_slice(NCH - NBUF + bx), os_[bx]).wait()

    return gather_k


_gather = _make_gather()


def _linear_body(x_ref, w_ref, b_ref, o_ref):
    o_ref[...] = (
        jnp.dot(x_ref[...], w_ref[...], preferred_element_type=jnp.float32)
        + b_ref[...]
    )


def _linear(image, W, b2d):
    blk = 512
    return pl.pallas_call(
        _linear_body,
        grid=(BATCH // blk,),
        in_specs=[
            pl.BlockSpec((blk, DIM), lambda i: (i, 0)),
            pl.BlockSpec((DIM, DIM), lambda i: (0, 0)),
            pl.BlockSpec((1, DIM), lambda i: (0, 0)),
        ],
        out_specs=pl.BlockSpec((blk, DIM), lambda i: (i, 0)),
        out_shape=jax.ShapeDtypeStruct((BATCH, DIM), jnp.float32),
    )(image, W, b2d)


def kernel(text, image, table, W, b):
    idx = text.astype(jnp.int32).T.reshape(NW, NCH, CHUNK)
    rows = _gather(idx, table)
    text_embedding = rows.reshape(HIST, BATCH, DIM).transpose(1, 0, 2)
    image_embedding = _linear(image, W, b.reshape(1, DIM))
    return (text_embedding, image_embedding)


# speedup vs baseline: 1.0012x; 1.0012x over previous
import functools

import jax
import jax.numpy as jnp
from jax import lax
from jax.experimental import pallas as pl
from jax.experimental.pallas import tpu as pltpu
from jax.experimental.pallas import tpu_sc as plsc

BATCH = 4096
HIST = 50
DIM = 128
NC = 2
NS = 16
NW = NC * NS
N_IDX = BATCH * HIST
PER_W = N_IDX // NW
CHUNK = 64
NCH = PER_W // CHUNK


VOCAB = 1000
NBUF = 10
STAGE_T = 16
STAGE_R = 64


def _make_gather():
    mesh = plsc.VectorSubcoreMesh(core_axis_name="c", subcore_axis_name="s")

    @functools.partial(
        pl.kernel,
        mesh=mesh,
        out_type=jax.ShapeDtypeStruct((N_IDX, DIM), jnp.float32),
        scratch_types=[
            pltpu.VMEM((NCH, CHUNK), jnp.int32),
        ] + [pltpu.VMEM((CHUNK, DIM), jnp.float32) for _ in range(NBUF)] + [
            pltpu.VMEM_SHARED((VOCAB, DIM), jnp.float32),
        ] + [pltpu.SemaphoreType.DMA for _ in range(2 * NBUF)],
    )
    def gather_k(idx_hbm, table_hbm, out_hbm, idx_v, *rest):
        bufs = rest[:NBUF]
        tab_s = rest[NBUF]
        gs = rest[NBUF + 1:2 * NBUF + 1]
        os_ = rest[2 * NBUF + 1:]
        sid = lax.axis_index("s")
        wid = sid * NC + lax.axis_index("c")
        base = wid * PER_W

        for t in range(STAGE_T):
            nrows = min(STAGE_R, VOCAB - t * STAGE_R)

            @pl.when(sid == t)
            def _(t=t, nrows=nrows):
                pltpu.sync_copy(table_hbm.at[pl.ds(t * STAGE_R, nrows)],
                                tab_s.at[pl.ds(t * STAGE_R, nrows)])

        pltpu.sync_copy(idx_hbm.at[wid], idx_v)
        plsc.subcore_barrier()

        def out_slice(c):
            return out_hbm.at[pl.ds(base + c * CHUNK, CHUNK)]

        for bx in range(NBUF):
            pltpu.async_copy(tab_s.at[idx_v.at[bx]], bufs[bx], gs[bx])

        def step(i, carry):
            c0 = NBUF * i
            for bx in range(NBUF):
                pltpu.make_async_copy(
                    tab_s.at[idx_v.at[c0]], bufs[bx], gs[bx]).wait()
                pltpu.async_copy(bufs[bx], out_slice(c0 + bx), os_[bx])

            @pl.when(i < NCH // NBUF - 1)
            def _():
                for bx in range(NBUF):
                    pltpu.make_async_copy(
                        bufs[bx], out_slice(c0 + bx), os_[bx]).wait()
                    pltpu.async_copy(
                        tab_s.at[idx_v.at[c0 + NBUF + bx]], bufs[bx], gs[bx])

            return carry

        lax.fori_loop(0, NCH // NBUF, step, 0, unroll=False)

        for bx in range(NBUF):
            pltpu.make_async_copy(
                bufs[bx], out_slice(NCH - NBUF + bx), os_[bx]).wait()

    return gather_k


_gather = _make_gather()


def _linear_body(x_ref, w_ref, b_ref, o_ref):
    o_ref[...] = (
        jnp.dot(x_ref[...], w_ref[...], preferred_element_type=jnp.float32)
        + b_ref[...]
    )


def _linear(image, W, b2d):
    blk = 512
    return pl.pallas_call(
        _linear_body,
        grid=(BATCH // blk,),
        in_specs=[
            pl.BlockSpec((blk, DIM), lambda i: (i, 0)),
            pl.BlockSpec((DIM, DIM), lambda i: (0, 0)),
            pl.BlockSpec((1, DIM), lambda i: (0, 0)),
        ],
        out_specs=pl.BlockSpec((blk, DIM), lambda i: (i, 0)),
        out_shape=jax.ShapeDtypeStruct((BATCH, DIM), jnp.float32),
    )(image, W, b2d)


def kernel(text, image, table, W, b):
    idx = text.astype(jnp.int32).T.reshape(NW, NCH, CHUNK)
    rows = _gather(idx, table)
    text_embedding = rows.reshape(HIST, BATCH, DIM).transpose(1, 0, 2)
    image_embedding = _linear(image, W, b.reshape(1, DIM))
    return (text_embedding, image_embedding)
